# baseline (speedup 1.0000x reference)
import functools
import jax
import jax.numpy as jnp
from jax import lax
from jax.experimental import pallas as pl
from jax.experimental.pallas import tpu as pltpu
from jax.experimental.pallas import tpu_sc as plsc

D = 256
H = 4
Ch = 64
N = 1024
BLK = 128
E = 32768

_NSUB = 16
_EPT = E // _NSUB
_CHUNK = 128
_NCHUNK = _EPT // _CHUNK
_TRASH = 8192


_SLICE = 256 * N * H
_TRASH = _SLICE
_ZBUF = 16384


def _adj_body(src_ref, dst_ref, alpha_ref, out_ref,
              src_v, dst_v, alpha_v, valbuf, idxbuf, zbuf, acc):
    c = lax.axis_index("c")
    s = lax.axis_index("s")
    base = s * _EPT
    pltpu.sync_copy(src_ref.at[pl.ds(base, _EPT)], src_v)
    pltpu.sync_copy(dst_ref.at[pl.ds(base, _EPT)], dst_v)
    pltpu.sync_copy(alpha_ref.at[:, pl.ds(base, _EPT)], alpha_v)
    zero16 = jnp.zeros((16,), jnp.float32)

    def zero_z(j, _):
        zbuf[pl.ds(j * 16, 16)] = zero16
        return 0

    lax.fori_loop(0, _ZBUF // 16, zero_z, 0)

    for p_loc in range(2):
        pass_id = c * 2 + p_loc
        for q in range(4):
            pltpu.sync_copy(
                zbuf, acc.at[pl.ds(s * (4 * _ZBUF) + q * _ZBUF, _ZBUF)])

        @pl.when(s == 0)
        def _():
            pltpu.sync_copy(zbuf.at[pl.ds(0, 128)],
                            acc.at[pl.ds(_TRASH, 128)])

        plsc.subcore_barrier()

        def do_chunk(ci, _):
            e0 = ci * _CHUNK
            for j in range(8):
                eo = e0 + j * 16
                sv = src_v[pl.ds(eo, 16)]
                dv = dst_v[pl.ds(eo, 16)]
                match = (sv >> 8) == pass_id
                flat0 = ((sv & 255) << 12) | (dv << 2)
                for h in range(4):
                    q = j * 64 + h * 16
                    idxbuf[q >> 7, pl.ds(q & 127, 16)] = jnp.where(
                        match, flat0 + h, _TRASH)
                    valbuf[pl.ds(q, 16)] = alpha_v[h, pl.ds(eo, 16)]
            for r in range(4):
                pltpu.sync_copy(valbuf.at[pl.ds(r * 128, 128)],
                                acc.at[idxbuf.at[r]], add=True)
            return 0

        lax.fori_loop(0, _NCHUNK, do_chunk, 0)
        plsc.subcore_barrier()
        pltpu.sync_copy(
            acc.at[pl.ds(s * (4 * _ZBUF), 4 * _ZBUF)],
            out_ref.at[pl.ds(pass_id * _SLICE + s * (4 * _ZBUF), 4 * _ZBUF)])
        plsc.subcore_barrier()


_adj_sc = pl.kernel(
    _adj_body,
    out_type=jax.ShapeDtypeStruct((N * N * H,), jnp.float32),
    mesh=plsc.VectorSubcoreMesh(core_axis_name="c", subcore_axis_name="s"),
    scratch_types=[
        pltpu.VMEM((_EPT,), jnp.int32),
        pltpu.VMEM((_EPT,), jnp.int32),
        pltpu.VMEM((4, _EPT), jnp.float32),
        pltpu.VMEM((512,), jnp.float32),
        pltpu.VMEM((4, 128), jnp.int32),
        pltpu.VMEM((_ZBUF,), jnp.float32),
        pltpu.VMEM_SHARED((_SLICE + 128,), jnp.float32),
    ],
)


def _dense_adj_sc(src, dst, alpha):
    part = _adj_sc(src, dst, alpha.T)
    return part.reshape(N, N, H)


def _att_pool_body(feat_ref, q_ref, Wq_ref, bq_ref, v_ref, mask_ref,
                   w_ref, aggr_ref):
    feat = feat_ref[...]
    q = q_ref[...]
    Wq = Wq_ref[...]
    bq = bq_ref[...]
    v = v_ref[...]
    mask = mask_ref[...]
    h = lax.dot_general(feat, Wq, (((2,), (0,)), ((), ())),
                        preferred_element_type=jnp.float32)
    h = jnp.tanh(h + bq[0][None, None, :] + q[0][None, None, :])
    scores = jnp.sum(h * v[0][None, None, :], axis=2)
    scores = jnp.where(mask > 0.5, scores, -1e9)
    m = jnp.max(scores, axis=1, keepdims=True)
    ex = jnp.exp(scores - m)
    w = ex / jnp.sum(ex, axis=1, keepdims=True)
    w_ref[...] = w
    aggr_ref[...] = jnp.sum(feat * w[:, :, None], axis=1)


def _att_pool(feat_p, q, Wq, bq, v, mask_p, T_pad):
    Bp = feat_p.shape[0]
    grid = (Bp // BLK,)
    return pl.pallas_call(
        _att_pool_body,
        grid=grid,
        in_specs=[
            pl.BlockSpec((BLK, T_pad, D), lambda i: (i, 0, 0)),
            pl.BlockSpec((1, 64), lambda i: (0, 0)),
            pl.BlockSpec((D, 64), lambda i: (0, 0)),
            pl.BlockSpec((1, 64), lambda i: (0, 0)),
            pl.BlockSpec((1, 64), lambda i: (0, 0)),
            pl.BlockSpec((BLK, T_pad), lambda i: (i, 0)),
        ],
        out_specs=[
            pl.BlockSpec((BLK, T_pad), lambda i: (i, 0)),
            pl.BlockSpec((BLK, D), lambda i: (i, 0)),
        ],
        out_shape=[
            jax.ShapeDtypeStruct((Bp, T_pad), jnp.float32),
            jax.ShapeDtypeStruct((Bp, D), jnp.float32),
        ],
    )(feat_p, q, Wq, bq, v, mask_p)


def _gatv2_edge(x, src, dst, Wl, Wr, att, b):
    xl = (x @ Wl).reshape(N, H, Ch)
    xr = (x @ Wr).reshape(N, H, Ch)
    e = jax.nn.leaky_relu(xl[src] + xr[dst], negative_slope=0.2)
    logit = jnp.sum(e * att[None, :, :], axis=-1)
    m = jax.ops.segment_max(logit, dst, num_segments=N)
    m = jnp.where(jnp.isfinite(m), m, 0.0)
    ex = jnp.exp(logit - m[dst])
    s = jax.ops.segment_sum(ex, dst, num_segments=N)
    alpha = ex / (s[dst] + 1e-16)
    out = jax.ops.segment_sum(xl[src] * alpha[:, :, None], dst, num_segments=N)
    return out.reshape(N, H * Ch) + b, alpha


def kernel(goal_feature, cap_feature, img_feature, Wg_w, Wg_b, end_w, end_b,
           ia_Wq, ia_bq, ia_Wk, ia_bk, ia_v, ca_Wq, ca_bq, ca_Wk, ca_bk, ca_v,
           c0_Wl, c0_Wr, c0_att, c0_b, c1_Wl, c1_Wr, c1_att, c1_b,
           cap_emb_mask, edge_index):
    convs = [(c0_Wl, c0_Wr, c0_att, c0_b), (c1_Wl, c1_Wr, c1_att, c1_b)]
    src = edge_index[0]
    dst = edge_index[1]
    B = cap_feature.shape[0]
    Bp = N

    def prep(feat, mask):
        T = feat.shape[1]
        T_pad = ((T + 63) // 64) * 64
        feat_p = jnp.zeros((Bp, T_pad, D), jnp.float32).at[:B, :T].set(feat)
        m = jnp.zeros((Bp, T_pad), jnp.float32)
        if mask is None:
            m = m.at[:B, :T].set(1.0)
        else:
            m = m.at[:B, :T].set(mask.astype(jnp.float32))
        return feat_p, m, T, T_pad

    def branch(feat, Wq, bq, Wk, bk, v, mask):
        feat_p, mask_p, T, T_pad = prep(feat, mask)
        bq2 = bq[None, :]
        v2 = v[None, :]
        w_list = []
        adj_list = []
        node = None
        for idx in range(2):
            if idx == 0:
                cg = goal_feature @ Wg_w + Wg_b
                ce = goal_feature @ end_w + end_b
            else:
                cg = node[0:1]
                ce = node[1:2]
            q = cg @ Wk + bk[None, :]
            w_p, aggr_p = _att_pool(feat_p, q, Wq, bq2, v2, mask_p, T_pad)
            w = w_p[:B, :T]
            aggr = aggr_p[:B]
            prev = jnp.concatenate([cg, ce, aggr], axis=0)
            out, alpha = _gatv2_edge(prev, src, dst, *convs[idx])
            node = jax.nn.elu(out + prev)
            w_list.append(w)
            adj_list.append(_dense_adj_sc(src, dst, alpha))
        return jnp.stack(w_list), jnp.stack(adj_list)

    img_w, img_adj = branch(img_feature, ia_Wq, ia_bq, ia_Wk, ia_bk, ia_v, None)
    cap_w, cap_adj = branch(cap_feature, ca_Wq, ca_bq, ca_Wk, ca_bk, ca_v,
                            cap_emb_mask)
    return (img_w, img_adj, cap_w, cap_adj)
